# baseline (speedup 1.0000x reference)
import functools

import jax
import jax.numpy as jnp
from jax import lax
from jax.experimental import pallas as pl
from jax.experimental.pallas import tpu as pltpu
from jax.experimental.pallas import tpu_sc as plsc

_EMBED = 128
_NSUB = 16


def _gather_side(table, idx_hbm, out_hbm, idx_v, buf0, buf1, gs0, gs1,
                 sid, b_per_w):
    base = sid * b_per_w
    pltpu.sync_copy(idx_hbm.at[pl.ds(base, b_per_w)], idx_v)

    pltpu.async_copy(table.at[idx_v.at[0]], buf0, gs0)
    pltpu.async_copy(table.at[idx_v.at[1]], buf1, gs1)

    def body(i, carry):
        j = 2 * i

        pltpu.make_async_copy(table.at[idx_v.at[j]], buf0, gs0).wait()
        pltpu.sync_copy(buf0, out_hbm.at[base + j])

        @pl.when(j + 2 < b_per_w)
        def _():
            pltpu.async_copy(table.at[idx_v.at[j + 2]], buf0, gs0)

        pltpu.make_async_copy(table.at[idx_v.at[j + 1]], buf1, gs1).wait()
        pltpu.sync_copy(buf1, out_hbm.at[base + j + 1])

        @pl.when(j + 3 < b_per_w)
        def _():
            pltpu.async_copy(table.at[idx_v.at[j + 3]], buf1, gs1)

        return carry

    lax.fori_loop(0, b_per_w // 2, body, 0)


@functools.lru_cache(maxsize=None)
def _build_side(batch, seq, which_core):
    assert batch % (_NSUB * 2) == 0
    b_per_w = batch // _NSUB
    mesh = plsc.VectorSubcoreMesh(core_axis_name="c", subcore_axis_name="s")

    @functools.partial(
        pl.kernel,
        out_type=jax.ShapeDtypeStruct((batch, seq, _EMBED), jnp.float32),
        scratch_types=[
            pltpu.VMEM((b_per_w, seq), jnp.int32),
            pltpu.VMEM((seq, _EMBED), jnp.float32),
            pltpu.VMEM((seq, _EMBED), jnp.float32),
            pltpu.SemaphoreType.DMA,
            pltpu.SemaphoreType.DMA,
        ],
        mesh=mesh,
    )
    def emb_kernel(table, idx, out, idx_v, buf0, buf1, gs0, gs1):
        cid = lax.axis_index("c")
        sid = lax.axis_index("s")

        @pl.when(cid == which_core)
        def _():
            _gather_side(table, idx, out, idx_v, buf0, buf1,
                         gs0, gs1, sid, b_per_w)

    return emb_kernel


def kernel(src_table, tgt_table, src_tokens, tgt_tokens):
    b, s = src_tokens.shape
    src_out = _build_side(b, s, 0)(src_table, src_tokens.astype(jnp.int32))
    tgt_out = _build_side(b, s, 1)(tgt_table, tgt_tokens.astype(jnp.int32))
    return (src_out, tgt_out)


# speedup vs baseline: 1.3434x; 1.3434x over previous
import functools

import jax
import jax.numpy as jnp
from jax import lax
from jax.experimental import pallas as pl
from jax.experimental.pallas import tpu as pltpu
from jax.experimental.pallas import tpu_sc as plsc

_EMBED = 128
_NSUB = 16
_NCORE = 2


def _gather_side(table, idx_hbm, out_hbm, idx_v, buf0, buf1, gs0, gs1,
                 wid, b_per_w):
    base = wid * b_per_w
    pltpu.sync_copy(idx_hbm.at[pl.ds(base, b_per_w)], idx_v)

    pltpu.async_copy(table.at[idx_v.at[0]], buf0, gs0)
    pltpu.async_copy(table.at[idx_v.at[1]], buf1, gs1)

    def body(i, carry):
        j = 2 * i

        pltpu.make_async_copy(table.at[idx_v.at[j]], buf0, gs0).wait()
        pltpu.sync_copy(buf0, out_hbm.at[base + j])

        @pl.when(j + 2 < b_per_w)
        def _():
            pltpu.async_copy(table.at[idx_v.at[j + 2]], buf0, gs0)

        pltpu.make_async_copy(table.at[idx_v.at[j + 1]], buf1, gs1).wait()
        pltpu.sync_copy(buf1, out_hbm.at[base + j + 1])

        @pl.when(j + 3 < b_per_w)
        def _():
            pltpu.async_copy(table.at[idx_v.at[j + 3]], buf1, gs1)

        return carry

    lax.fori_loop(0, b_per_w // 2, body, 0)


@functools.lru_cache(maxsize=None)
def _build_side(batch, seq):
    n_workers = _NCORE * _NSUB
    assert batch % (n_workers * 2) == 0
    b_per_w = batch // n_workers
    mesh = plsc.VectorSubcoreMesh(core_axis_name="c", subcore_axis_name="s")

    @functools.partial(
        pl.kernel,
        out_type=jax.ShapeDtypeStruct((batch, seq, _EMBED), jnp.float32),
        scratch_types=[
            pltpu.VMEM((b_per_w, seq), jnp.int32),
            pltpu.VMEM((seq, _EMBED), jnp.float32),
            pltpu.VMEM((seq, _EMBED), jnp.float32),
            pltpu.SemaphoreType.DMA,
            pltpu.SemaphoreType.DMA,
        ],
        mesh=mesh,
    )
    def emb_kernel(table, idx, out, idx_v, buf0, buf1, gs0, gs1):
        cid = lax.axis_index("c")
        sid = lax.axis_index("s")
        wid = cid * _NSUB + sid
        _gather_side(table, idx, out, idx_v, buf0, buf1, gs0, gs1,
                     wid, b_per_w)

    return emb_kernel


def kernel(src_table, tgt_table, src_tokens, tgt_tokens):
    b, s = src_tokens.shape
    side = _build_side(b, s)
    src_out = side(src_table, src_tokens.astype(jnp.int32))
    tgt_out = side(tgt_table, tgt_tokens.astype(jnp.int32))
    return (src_out, tgt_out)


# speedup vs baseline: 2.3142x; 1.7226x over previous
import functools

import jax
import jax.numpy as jnp
from jax import lax
from jax.experimental import pallas as pl
from jax.experimental.pallas import tpu as pltpu
from jax.experimental.pallas import tpu_sc as plsc

_EMBED = 128
_NSUB = 16
_CHUNK = 80


def _gather_side(table, idx_hbm, out_hbm, idx_v, buf0, buf1, gs0, gs1,
                 sid, n_per_w):
    base = sid * n_per_w
    n_chunks = n_per_w // _CHUNK
    pltpu.sync_copy(idx_hbm.at[pl.ds(base, n_per_w)], idx_v)

    pltpu.async_copy(table.at[idx_v.at[pl.ds(0, _CHUNK)]], buf0, gs0)
    pltpu.async_copy(table.at[idx_v.at[pl.ds(_CHUNK, _CHUNK)]], buf1, gs1)

    def body(i, carry):
        j = 2 * i

        pltpu.make_async_copy(
            table.at[idx_v.at[pl.ds(j * _CHUNK, _CHUNK)]], buf0, gs0).wait()
        pltpu.sync_copy(buf0, out_hbm.at[pl.ds(base + j * _CHUNK, _CHUNK)])

        @pl.when(j + 2 < n_chunks)
        def _():
            pltpu.async_copy(
                table.at[idx_v.at[pl.ds((j + 2) * _CHUNK, _CHUNK)]],
                buf0, gs0)

        pltpu.make_async_copy(
            table.at[idx_v.at[pl.ds((j + 1) * _CHUNK, _CHUNK)]],
            buf1, gs1).wait()
        pltpu.sync_copy(
            buf1, out_hbm.at[pl.ds(base + (j + 1) * _CHUNK, _CHUNK)])

        @pl.when(j + 3 < n_chunks)
        def _():
            pltpu.async_copy(
                table.at[idx_v.at[pl.ds((j + 3) * _CHUNK, _CHUNK)]],
                buf1, gs1)

        return carry

    lax.fori_loop(0, n_chunks // 2, body, 0)


@functools.lru_cache(maxsize=None)
def _build(n_rows):
    assert n_rows % (_NSUB * _CHUNK * 2) == 0
    n_per_w = n_rows // _NSUB
    mesh = plsc.VectorSubcoreMesh(core_axis_name="c", subcore_axis_name="s")

    @functools.partial(
        pl.kernel,
        out_type=[
            jax.ShapeDtypeStruct((n_rows, _EMBED), jnp.float32),
            jax.ShapeDtypeStruct((n_rows, _EMBED), jnp.float32),
        ],
        scratch_types=[
            pltpu.VMEM((n_per_w,), jnp.int32),
            pltpu.VMEM((_CHUNK, _EMBED), jnp.float32),
            pltpu.VMEM((_CHUNK, _EMBED), jnp.float32),
            pltpu.SemaphoreType.DMA,
            pltpu.SemaphoreType.DMA,
        ],
        mesh=mesh,
    )
    def emb_kernel(src_table, tgt_table, src_idx, tgt_idx,
                   src_out, tgt_out, idx_v, buf0, buf1, gs0, gs1):
        cid = lax.axis_index("c")
        sid = lax.axis_index("s")

        @pl.when(cid == 0)
        def _():
            _gather_side(src_table, src_idx, src_out, idx_v, buf0, buf1,
                         gs0, gs1, sid, n_per_w)

        @pl.when(cid == 1)
        def _():
            _gather_side(tgt_table, tgt_idx, tgt_out, idx_v, buf0, buf1,
                         gs0, gs1, sid, n_per_w)

    return emb_kernel


def kernel(src_table, tgt_table, src_tokens, tgt_tokens):
    b, s = src_tokens.shape
    n_rows = b * s
    src_idx = src_tokens.astype(jnp.int32).T.reshape(-1)
    tgt_idx = tgt_tokens.astype(jnp.int32).T.reshape(-1)
    src_out, tgt_out = _build(n_rows)(src_table, tgt_table, src_idx, tgt_idx)
    return (
        src_out.reshape(s, b, _EMBED).transpose(1, 0, 2),
        tgt_out.reshape(s, b, _EMBED).transpose(1, 0, 2),
    )
